# baseline (speedup 1.0000x reference)
import functools

import numpy as np
import jax
import jax.numpy as jnp
from jax import lax
from jax.experimental import pallas as pl
from jax.experimental.pallas import tpu as pltpu
from jax.experimental.pallas import tpu_sc as plsc

_VOCAB = 100000
_D = 512
_B = 4
_S = 4096

_NC = 2
_NS = 16
_NW = _NC * _NS
_SPW = _S // _NW
_C = 16
_NH = _SPW // _C
_G = _B * _C


def _pe_values(x) -> jnp.ndarray:
    zero = jnp.minimum(x[0, 0], 0).astype(jnp.float32)
    pos = jnp.arange(_S, dtype=jnp.float32)[:, None] + zero
    div = jnp.exp(jnp.arange(0, _D, 2, dtype=jnp.float32)
                  * (-np.log(10000.0) / _D))
    ang = pos * div[None, :]
    return jnp.stack([jnp.sin(ang), jnp.cos(ang)], axis=-1).reshape(_S, _D)


@functools.partial(
    pl.kernel,
    out_type=jax.ShapeDtypeStruct((_B, _S, _D), jnp.float32),
    mesh=plsc.VectorSubcoreMesh(core_axis_name="c", subcore_axis_name="s"),
    scratch_types=[
        pltpu.VMEM((_B, _SPW), jnp.int32),
        pltpu.VMEM((_NH, _G), jnp.int32),
        pltpu.VMEM((2, _C, _D), jnp.float32),
        pltpu.VMEM((2, _G, _D), jnp.float32),
        pltpu.SemaphoreType.DMA,
        pltpu.SemaphoreType.DMA,
        pltpu.SemaphoreType.DMA,
        pltpu.SemaphoreType.DMA,
        pltpu.SemaphoreType.DMA,
        pltpu.SemaphoreType.DMA,
        pltpu.SemaphoreType.DMA,
        pltpu.SemaphoreType.DMA,
        pltpu.SemaphoreType.DMA,
        pltpu.SemaphoreType.DMA,
        pltpu.SemaphoreType.DMA,
        pltpu.SemaphoreType.DMA,
    ],
)
def _embed(x_hbm, pe_hbm, table_hbm, out_hbm, x_stage, idx_all, pe_v, rows,
           pp0, pp1, gg0, gg1, o00, o01, o02, o03, o10, o11, o12, o13):
    wid = lax.axis_index("s") * _NC + lax.axis_index("c")
    s_base = wid * _SPW
    psem = (pp0, pp1)
    gsem = (gg0, gg1)
    osem = ((o00, o01, o02, o03), (o10, o11, o12, o13))
    pf = [None, None]
    ga = [None, None]
    st = [[None] * _B, [None] * _B]

    for b in range(_B):
        pltpu.sync_copy(x_hbm.at[b, pl.ds(s_base, _SPW)], x_stage.at[b])
    for h in range(_NH):
        for b in range(_B):
            idx_all[h, pl.ds(b * _C, _C)] = x_stage[b, pl.ds(h * _C, _C)]

    def load(h):
        g = h % 2
        for b in range(_B):
            if st[g][b] is not None:
                st[g][b].wait()
        pf[g] = pltpu.async_copy(
            pe_hbm.at[pl.ds(s_base + h * _C, _C)], pe_v.at[g], psem[g])
        ga[g] = pltpu.async_copy(
            table_hbm.at[idx_all.at[h]], rows.at[g], gsem[g])

    load(0)
    for h in range(_NH):
        g = h % 2
        if h + 1 < _NH:
            load(h + 1)
        pf[g].wait()
        ga[g].wait()

        def _row(i, _):
            for j in range(_D // 16):
                sl = pl.ds(j * 16, 16)
                v = pe_v[g, i, sl]
                for b in range(_B):
                    plsc.addupdate(rows.at[g, b * _C + i, sl], v)
            return 0

        lax.fori_loop(0, _C, _row, 0)
        for b in range(_B):
            st[g][b] = pltpu.async_copy(
                rows.at[g, pl.ds(b * _C, _C)],
                out_hbm.at[b, pl.ds(s_base + h * _C, _C)], osem[g][b])
    for slot in st:
        for w in slot:
            if w is not None:
                w.wait()


def kernel(x, table):
    x = x.astype(jnp.int32)
    pe = _pe_values(x)
    return _embed(x, pe, table)


# speedup vs baseline: 1.7146x; 1.7146x over previous
import functools

import numpy as np
import jax
import jax.numpy as jnp
from jax import lax
from jax.experimental import pallas as pl
from jax.experimental.pallas import tpu as pltpu
from jax.experimental.pallas import tpu_sc as plsc

_VOCAB = 100000
_D = 512
_B = 4
_S = 4096

_NC = 2
_NS = 16
_NW = _NC * _NS
_SPW = _S // _NW
_C = 16
_NH = _SPW // _C
_G = _B * _C


def _pe_table() -> np.ndarray:
    pos = np.arange(_S, dtype=np.float64)[:, None]
    div = np.exp(np.arange(0, _D, 2, dtype=np.float64) * (-np.log(10000.0) / _D))
    pe = np.zeros((_S, _D), np.float64)
    pe[:, 0::2] = np.sin(pos * div)
    pe[:, 1::2] = np.cos(pos * div)
    return pe.astype(np.float16)


_PE = _pe_table()


def _pe_values(x) -> jnp.ndarray:
    zero = jnp.minimum(x[0, 0], 0).astype(jnp.float16)
    return (jnp.asarray(_PE) + zero).astype(jnp.float32)


@functools.partial(
    pl.kernel,
    out_type=jax.ShapeDtypeStruct((_B, _S, _D), jnp.float32),
    mesh=plsc.VectorSubcoreMesh(core_axis_name="c", subcore_axis_name="s"),
    scratch_types=[
        pltpu.VMEM((_B, _SPW), jnp.int32),
        pltpu.VMEM((_NH, _G), jnp.int32),
        pltpu.VMEM((2, _C, _D), jnp.float32),
        pltpu.VMEM((2, _G, _D), jnp.float32),
        pltpu.SemaphoreType.DMA,
        pltpu.SemaphoreType.DMA,
        pltpu.SemaphoreType.DMA,
        pltpu.SemaphoreType.DMA,
        pltpu.SemaphoreType.DMA,
        pltpu.SemaphoreType.DMA,
        pltpu.SemaphoreType.DMA,
        pltpu.SemaphoreType.DMA,
        pltpu.SemaphoreType.DMA,
        pltpu.SemaphoreType.DMA,
        pltpu.SemaphoreType.DMA,
        pltpu.SemaphoreType.DMA,
    ],
)
def _embed(x_hbm, pe_hbm, table_hbm, out_hbm, x_stage, idx_all, pe_v, rows,
           pp0, pp1, gg0, gg1, o00, o01, o02, o03, o10, o11, o12, o13):
    wid = lax.axis_index("s") * _NC + lax.axis_index("c")
    s_base = wid * _SPW
    psem = (pp0, pp1)
    gsem = (gg0, gg1)
    osem = ((o00, o01, o02, o03), (o10, o11, o12, o13))
    pf = [None, None]
    ga = [None, None]
    st = [[None] * _B, [None] * _B]

    for b in range(_B):
        pltpu.sync_copy(x_hbm.at[b, pl.ds(s_base, _SPW)], x_stage.at[b])
    for h in range(_NH):
        for b in range(_B):
            idx_all[h, pl.ds(b * _C, _C)] = x_stage[b, pl.ds(h * _C, _C)]

    def load(h):
        g = h % 2
        for b in range(_B):
            if st[g][b] is not None:
                st[g][b].wait()
        pf[g] = pltpu.async_copy(
            pe_hbm.at[pl.ds(s_base + h * _C, _C)], pe_v.at[g], psem[g])
        ga[g] = pltpu.async_copy(
            table_hbm.at[idx_all.at[h]], rows.at[g], gsem[g])

    load(0)
    for h in range(_NH):
        g = h % 2
        if h + 1 < _NH:
            load(h + 1)
        pf[g].wait()
        ga[g].wait()

        def _row(i, _):
            for j in range(_D // 16):
                sl = pl.ds(j * 16, 16)
                v = pe_v[g, i, sl]
                for b in range(_B):
                    plsc.addupdate(rows.at[g, b * _C + i, sl], v)
            return 0

        lax.fori_loop(0, _C, _row, 0)
        for b in range(_B):
            st[g][b] = pltpu.async_copy(
                rows.at[g, pl.ds(b * _C, _C)],
                out_hbm.at[b, pl.ds(s_base + h * _C, _C)], osem[g][b])
    for slot in st:
        for w in slot:
            if w is not None:
                w.wait()


def kernel(x, table):
    x = x.astype(jnp.int32)
    pe = _pe_values(x)
    return _embed(x, pe, table)


# speedup vs baseline: 1.7736x; 1.0344x over previous
import functools

import numpy as np
import jax
import jax.numpy as jnp
from jax import lax
from jax.experimental import pallas as pl
from jax.experimental.pallas import tpu as pltpu
from jax.experimental.pallas import tpu_sc as plsc

_VOCAB = 100000
_D = 512
_B = 4
_S = 4096

_NC = 2
_NS = 16
_NW = _NC * _NS
_SPW = _S // _NW
_C = 16
_NH = _SPW // _C
_G = _B * _C
_NSLOT = 3


def _pe_table() -> np.ndarray:
    pos = np.arange(_S, dtype=np.float32)[:, None]
    div = np.exp(np.arange(0, _D, 2, dtype=np.float32) * (-np.log(10000.0) / _D))
    pe = np.zeros((_S, _D), np.float32)
    pe[:, 0::2] = np.sin(pos * div)
    pe[:, 1::2] = np.cos(pos * div)
    return pe


_PE = _pe_table()


@functools.partial(
    pl.kernel,
    out_type=jax.ShapeDtypeStruct((_B, _S, _D), jnp.float32),
    mesh=plsc.VectorSubcoreMesh(core_axis_name="c", subcore_axis_name="s"),
    scratch_types=[
        pltpu.VMEM((_B, _SPW), jnp.int32),
        pltpu.VMEM((_NH, _G), jnp.int32),
        pltpu.VMEM((_NSLOT, _C, _D), jnp.float32),
        pltpu.VMEM((_NSLOT, _G, _D), jnp.float32),
        pltpu.SemaphoreType.DMA,
        pltpu.SemaphoreType.DMA,
        pltpu.SemaphoreType.DMA,
        pltpu.SemaphoreType.DMA,
        pltpu.SemaphoreType.DMA,
        pltpu.SemaphoreType.DMA,
        pltpu.SemaphoreType.DMA,
        pltpu.SemaphoreType.DMA,
        pltpu.SemaphoreType.DMA,
        pltpu.SemaphoreType.DMA,
        pltpu.SemaphoreType.DMA,
        pltpu.SemaphoreType.DMA,
        pltpu.SemaphoreType.DMA,
        pltpu.SemaphoreType.DMA,
        pltpu.SemaphoreType.DMA,
        pltpu.SemaphoreType.DMA,
        pltpu.SemaphoreType.DMA,
        pltpu.SemaphoreType.DMA,
    ],
)
def _embed(x_hbm, pe_hbm, table_hbm, out_hbm, x_stage, idx_all, pe_v, rows,
           pp0, pp1, pp2, gg0, gg1, gg2,
           o00, o01, o02, o03, o10, o11, o12, o13, o20, o21, o22, o23):
    wid = lax.axis_index("s") * _NC + lax.axis_index("c")
    s_base = wid * _SPW
    psem = (pp0, pp1, pp2)
    gsem = (gg0, gg1, gg2)
    osem = ((o00, o01, o02, o03), (o10, o11, o12, o13), (o20, o21, o22, o23))
    pf = [None] * _NSLOT
    ga = [None] * _NSLOT
    st = [[None] * _B for _ in range(_NSLOT)]

    for b in range(_B):
        pltpu.sync_copy(x_hbm.at[b, pl.ds(s_base, _SPW)], x_stage.at[b])
    for h in range(_NH):
        for b in range(_B):
            idx_all[h, pl.ds(b * _C, _C)] = x_stage[b, pl.ds(h * _C, _C)]

    def load(h):
        g = h % _NSLOT
        for b in range(_B):
            if st[g][b] is not None:
                st[g][b].wait()
        pf[g] = pltpu.async_copy(
            pe_hbm.at[pl.ds(s_base + h * _C, _C)], pe_v.at[g], psem[g])
        ga[g] = pltpu.async_copy(
            table_hbm.at[idx_all.at[h]], rows.at[g], gsem[g])

    load(0)
    load(1)
    for h in range(_NH):
        g = h % _NSLOT
        if h + 2 < _NH:
            load(h + 2)
        pf[g].wait()
        ga[g].wait()

        def _row(i, _):
            for j in range(_D // 16):
                sl = pl.ds(j * 16, 16)
                v = pe_v[g, i, sl]
                for b in range(_B):
                    plsc.addupdate(rows.at[g, b * _C + i, sl], v)
            return 0

        lax.fori_loop(0, _C, _row, 0)
        for b in range(_B):
            st[g][b] = pltpu.async_copy(
                rows.at[g, pl.ds(b * _C, _C)],
                out_hbm.at[b, pl.ds(s_base + h * _C, _C)], osem[g][b])
    for slot in st:
        for w in slot:
            if w is not None:
                w.wait()


def kernel(x, table):
    x = x.astype(jnp.int32)
    pe = jnp.asarray(_PE)
    return _embed(x, pe, table)


# speedup vs baseline: 1.8142x; 1.0229x over previous
import functools

import numpy as np
import jax
import jax.numpy as jnp
from jax import lax
from jax.experimental import pallas as pl
from jax.experimental.pallas import tpu as pltpu
from jax.experimental.pallas import tpu_sc as plsc

_VOCAB = 100000
_D = 512
_B = 4
_S = 4096

_NC = 2
_NS = 16
_NW = _NC * _NS
_SPW = _S // _NW
_C = 16
_NH = _SPW // _C
_G = _B * _C
_NSLOT = 3


def _pe_table() -> np.ndarray:
    pos = np.arange(_S, dtype=np.float32)[:, None]
    div = np.exp(np.arange(0, _D, 2, dtype=np.float32) * (-np.log(10000.0) / _D))
    pe = np.zeros((_S, _D), np.float32)
    pe[:, 0::2] = np.sin(pos * div)
    pe[:, 1::2] = np.cos(pos * div)
    return pe


_PE = _pe_table()


@functools.partial(
    pl.kernel,
    out_type=jax.ShapeDtypeStruct((_B, _S, _D), jnp.float32),
    mesh=plsc.VectorSubcoreMesh(core_axis_name="c", subcore_axis_name="s"),
    scratch_types=[
        pltpu.VMEM((_B, _SPW), jnp.int32),
        pltpu.VMEM((_NH, _G), jnp.int32),
        pltpu.VMEM((_NSLOT, _C, _D), jnp.float32),
        pltpu.VMEM((_NSLOT, _G, _D), jnp.float32),
        pltpu.SemaphoreType.DMA,
        pltpu.SemaphoreType.DMA,
        pltpu.SemaphoreType.DMA,
        pltpu.SemaphoreType.DMA,
        pltpu.SemaphoreType.DMA,
        pltpu.SemaphoreType.DMA,
        pltpu.SemaphoreType.DMA,
        pltpu.SemaphoreType.DMA,
        pltpu.SemaphoreType.DMA,
        pltpu.SemaphoreType.DMA,
        pltpu.SemaphoreType.DMA,
        pltpu.SemaphoreType.DMA,
        pltpu.SemaphoreType.DMA,
        pltpu.SemaphoreType.DMA,
        pltpu.SemaphoreType.DMA,
        pltpu.SemaphoreType.DMA,
        pltpu.SemaphoreType.DMA,
        pltpu.SemaphoreType.DMA,
    ],
)
def _embed(x_hbm, pe_hbm, table_hbm, out_hbm, x_stage, idx_all, pe_v, rows,
           pp0, pp1, pp2, gg0, gg1, gg2,
           o00, o01, o02, o03, o10, o11, o12, o13, o20, o21, o22, o23):
    wid = lax.axis_index("s") * _NC + lax.axis_index("c")
    s_base = wid * _SPW
    psem = (pp0, pp1, pp2)
    gsem = (gg0, gg1, gg2)
    osem = ((o00, o01, o02, o03), (o10, o11, o12, o13), (o20, o21, o22, o23))
    pf = [None] * _NSLOT
    ga = [None] * _NSLOT
    st = [[None] * _B for _ in range(_NSLOT)]

    xcp = [pltpu.async_copy(x_hbm.at[b, pl.ds(s_base, _SPW)], x_stage.at[b],
                            osem[0][b])
           for b in range(_B)]
    for c in xcp:
        c.wait()
    for h in range(_NH):
        for b in range(_B):
            idx_all[h, pl.ds(b * _C, _C)] = x_stage[b, pl.ds(h * _C, _C)]

    def load(h):
        g = h % _NSLOT
        pf[g] = pltpu.async_copy(
            pe_hbm.at[pl.ds(s_base + h * _C, _C)], pe_v.at[g], psem[g])
        for b in range(_B):
            if st[g][b] is not None:
                st[g][b].wait()
        ga[g] = pltpu.async_copy(
            table_hbm.at[idx_all.at[h]], rows.at[g], gsem[g])

    load(0)
    load(1)
    for h in range(_NH):
        g = h % _NSLOT
        if h + 2 < _NH:
            load(h + 2)
        pf[g].wait()
        ga[g].wait()

        def _row(i, _):
            for j in range(_D // 16):
                sl = pl.ds(j * 16, 16)
                v = pe_v[g, i, sl]
                for b in range(_B):
                    plsc.addupdate(rows.at[g, b * _C + i, sl], v)
            return 0

        lax.fori_loop(0, _C, _row, 0)
        for b in range(_B):
            st[g][b] = pltpu.async_copy(
                rows.at[g, pl.ds(b * _C, _C)],
                out_hbm.at[b, pl.ds(s_base + h * _C, _C)], osem[g][b])
    for slot in st:
        for w in slot:
            if w is not None:
                w.wait()


def kernel(x, table):
    x = x.astype(jnp.int32)
    pe = jnp.asarray(_PE)
    return _embed(x, pe, table)
